# Initial kernel scaffold; baseline (speedup 1.0000x reference)
import jax
import jax.numpy as jnp
from jax.experimental import pallas as pl


def kernel(h, incident_nodes, incident_edges, incident_values, degree_v_values, degree_e_values, sent_index, sent_values, layer, params):
    raise NotImplementedError("write your pallas kernel here")


# speedup vs baseline: 1.4342x; 1.4342x over previous
import functools

import jax
import jax.numpy as jnp
from jax import lax
from jax.experimental import pallas as pl
from jax.experimental.pallas import tpu as pltpu
from jax.experimental.pallas import tpu_sc as plsc

NW = 32
CHUNK = 128
D = 128


def _pack3(idx_src, idx_dst, w):
    nnz = idx_src.shape[0]
    nch = -(-nnz // (CHUNK * NW))
    tot = nch * NW * CHUNK
    pad = tot - nnz
    a = jnp.pad(idx_src.astype(jnp.int32), (0, pad))
    b = jnp.pad(idx_dst.astype(jnp.int32), (0, pad))
    P = jnp.stack([a, b]).reshape(2, nch * NW, CHUNK).transpose(1, 0, 2)
    W = jnp.pad(w, (0, pad)).reshape(nch * NW, 1, CHUNK)
    return P, W, nch


def _pack2(idx_a, idx_b):
    nnz = idx_a.shape[0]
    nch = -(-nnz // (CHUNK * NW))
    tot = nch * NW * CHUNK
    pad = tot - nnz
    a = jnp.pad(idx_a.astype(jnp.int32), (0, pad))
    b = jnp.pad(idx_b.astype(jnp.int32), (0, pad))
    P = jnp.stack([a, b]).reshape(2, nch * NW, CHUNK).transpose(1, 0, 2)
    return P, nch


def _bcast_lane(v, j):
    return lax.gather(
        v, jnp.full((16, 1), j, jnp.int32),
        lax.GatherDimensionNumbers(offset_dims=(), collapsed_slice_dims=(0,),
                                   start_index_map=(0,)),
        (1,), mode=lax.GatherScatterMode.PROMISE_IN_BOUNDS)

def _sc_spmm(table, P, W, nch, np_pad):
    TW = table.shape[1]
    out_type = jax.ShapeDtypeStruct((2, np_pad, TW), jnp.float32)
    scratch = [
        pltpu.VMEM((2, CHUNK), jnp.int32),
        pltpu.VMEM((1, CHUNK), jnp.float32),
        pltpu.VMEM((CHUNK, TW), jnp.float32),
        pltpu.VMEM_SHARED((np_pad, TW), jnp.float32),
        pltpu.SemaphoreType.DMA,
    ]
    mesh = plsc.VectorSubcoreMesh(core_axis_name="c", subcore_axis_name="s")

    def body(table_h, p_h, w_h, outp_h, pbuf, wbuf, rows, accp, sem):
        cid = lax.axis_index("c")
        sid = lax.axis_index("s")
        z16 = jnp.zeros((16,), jnp.float32)

        def zb(i, _):
            for r in range(TW // 16):
                rows[i, pl.ds(r * 16, 16)] = z16
            return 0
        lax.fori_loop(0, 128, zb, 0)

        rpt = np_pad // 16
        for s2 in range(rpt // 128):
            r0 = sid * rpt + s2 * 128
            pltpu.sync_copy(rows, accp.at[pl.ds(r0, 128)])
        plsc.subcore_barrier()

        def chunk(i, _):
            ch = (cid * 16 + sid) * nch + i
            pltpu.sync_copy(p_h.at[ch], pbuf)
            pltpu.sync_copy(w_h.at[ch], wbuf)
            pltpu.async_copy(table_h.at[pbuf.at[0]], rows, sem).wait()

            def scale(j16, _):
                w16 = wbuf[0, pl.ds(j16 * 16, 16)]
                for j in range(16):
                    wv = _bcast_lane(w16, j)
                    row = j16 * 16 + j
                    for r in range(TW // 16):
                        rows[row, pl.ds(r * 16, 16)] = (
                            rows[row, pl.ds(r * 16, 16)] * wv)
                return 0
            lax.fori_loop(0, CHUNK // 16, scale, 0)
            pltpu.sync_copy(rows, accp.at[pbuf.at[1]], add=True)
            return 0
        lax.fori_loop(0, nch, chunk, 0)
        plsc.subcore_barrier()

        for s2 in range(rpt // 128):
            r0 = sid * rpt + s2 * 128
            pltpu.sync_copy(accp.at[pl.ds(r0, 128)], outp_h.at[cid, pl.ds(r0, 128)])

    k = pl.kernel(body, out_type=out_type, mesh=mesh, scratch_types=scratch)
    return k(table, P, W)


def _sc_rowsum(P, W, nch, np_pad):
    out_type = jax.ShapeDtypeStruct((2, np_pad, D), jnp.float32)
    scratch = [
        pltpu.VMEM((2, CHUNK), jnp.int32),
        pltpu.VMEM((1, CHUNK), jnp.float32),
        pltpu.VMEM((CHUNK, D), jnp.float32),
        pltpu.VMEM_SHARED((np_pad, D), jnp.float32),
        pltpu.SemaphoreType.DMA,
    ]
    mesh = plsc.VectorSubcoreMesh(core_axis_name="c", subcore_axis_name="s")

    def body(p_h, w_h, outp_h, pbuf, wbuf, rows, accp, sem):
        cid = lax.axis_index("c")
        sid = lax.axis_index("s")
        z16 = jnp.zeros((16,), jnp.float32)
        i16 = lax.iota(jnp.int32, 16)

        def zb(i, _):
            for r in range(D // 16):
                rows[i, pl.ds(r * 16, 16)] = z16
            return 0
        lax.fori_loop(0, 128, zb, 0)

        rpt = np_pad // 16
        for s2 in range(rpt // 128):
            r0 = sid * rpt + s2 * 128
            pltpu.sync_copy(rows, accp.at[pl.ds(r0, 128)])
        plsc.subcore_barrier()

        def chunk(i, _):
            ch = (cid * 16 + sid) * nch + i
            pltpu.sync_copy(p_h.at[ch], pbuf)
            pltpu.sync_copy(w_h.at[ch], wbuf)

            def scale(j16, _):
                w16 = wbuf[0, pl.ds(j16 * 16, 16)]
                for j in range(16):
                    wv = _bcast_lane(w16, j)
                    rows[j16 * 16 + j, pl.ds(0, 16)] = jnp.where(i16 == 0, wv, 0.0)
                return 0
            lax.fori_loop(0, CHUNK // 16, scale, 0)
            pltpu.sync_copy(rows, accp.at[pbuf.at[1]], add=True)
            return 0
        lax.fori_loop(0, nch, chunk, 0)
        plsc.subcore_barrier()

        for s2 in range(rpt // 128):
            r0 = sid * rpt + s2 * 128
            pltpu.sync_copy(accp.at[pl.ds(r0, 128)], outp_h.at[cid, pl.ds(r0, 128)])

    k = pl.kernel(body, out_type=out_type, mesh=mesh, scratch_types=scratch)
    return k(P, W)


def _sc_gather_add(A, B, P, nch):
    tot = P.shape[0] * CHUNK
    W = A.shape[1]
    scratch = [
        pltpu.VMEM((2, CHUNK), jnp.int32),
        pltpu.VMEM((CHUNK, W), jnp.float32),
        pltpu.VMEM((CHUNK, W), jnp.float32),
        pltpu.SemaphoreType.DMA,
    ]
    mesh = plsc.VectorSubcoreMesh(core_axis_name="c", subcore_axis_name="s")

    def body(a_h, b_h, p_h, g_h, pbuf, ga, gb, sem):
        cid = lax.axis_index("c")
        sid = lax.axis_index("s")

        def chunk(i, _):
            ch = (cid * 16 + sid) * nch + i
            pltpu.sync_copy(p_h.at[ch], pbuf)
            pltpu.async_copy(a_h.at[pbuf.at[0]], ga, sem).wait()
            pltpu.async_copy(b_h.at[pbuf.at[1]], gb, sem).wait()

            def add(j, _):
                for r in range(W // 16):
                    ga[j, pl.ds(r * 16, 16)] = (ga[j, pl.ds(r * 16, 16)]
                                                + gb[j, pl.ds(r * 16, 16)])
                return 0
            lax.fori_loop(0, CHUNK, add, 0)
            pltpu.sync_copy(ga, g_h.at[pl.ds(ch * CHUNK, CHUNK)])
            return 0
        lax.fori_loop(0, nch, chunk, 0)

    k = pl.kernel(body, out_type=jax.ShapeDtypeStruct((tot, W), jnp.float32),
                  mesh=mesh, scratch_types=scratch)
    return k(A, B, P)


def _tc(fn, out_shape, *args):
    return pl.pallas_call(fn, out_shape=out_shape)(*args)


def _mlp1_kernel(h_ref, w1_ref, b1_ref, w2_ref, b2_ref, wa_ref,
                 hm_ref, a1_ref):
    h = h_ref[...]
    t = jnp.maximum(jnp.dot(h, w1_ref[...], preferred_element_type=jnp.float32,
                            precision=lax.Precision.HIGHEST) + b1_ref[...], 0.0)
    hm = jnp.dot(t, w2_ref[...], preferred_element_type=jnp.float32,
                 precision=lax.Precision.HIGHEST) + b2_ref[...]
    hm_ref[...] = hm
    a1_ref[...] = jnp.dot(hm, wa_ref[...], preferred_element_type=jnp.float32,
                          precision=lax.Precision.HIGHEST)


def _mlp1(h, w1, b1, w2, b2, wa):
    n = h.shape[0]
    BR = 1000
    return pl.pallas_call(
        _mlp1_kernel,
        out_shape=(jax.ShapeDtypeStruct((n, D), jnp.float32),
                   jax.ShapeDtypeStruct((n, 2 * D), jnp.float32)),
        grid=(n // BR,),
        in_specs=[pl.BlockSpec((BR, D), lambda i: (i, 0)),
                  pl.BlockSpec((D, D), lambda i: (0, 0)),
                  pl.BlockSpec((1, D), lambda i: (0, 0)),
                  pl.BlockSpec((D, D), lambda i: (0, 0)),
                  pl.BlockSpec((1, D), lambda i: (0, 0)),
                  pl.BlockSpec((D, 2 * D), lambda i: (0, 0))],
        out_specs=(pl.BlockSpec((BR, D), lambda i: (i, 0)),
                   pl.BlockSpec((BR, 2 * D), lambda i: (i, 0))),
    )(h, w1, b1, w2, b2, wa)


def _finalize_b_kernel(p_ref, deg_ref, wb_ref, bb_ref, b_ref):
    ht = (p_ref[0] + p_ref[1]) * deg_ref[...]
    b_ref[...] = jnp.dot(ht, wb_ref[...],
                         preferred_element_type=jnp.float32,
                         precision=lax.Precision.HIGHEST) + bb_ref[...]


def _finalize_b(parts, deg, wb, bb):
    n = parts.shape[1]
    BR = 1000
    return pl.pallas_call(
        _finalize_b_kernel,
        out_shape=jax.ShapeDtypeStruct((n, 2 * D), jnp.float32),
        grid=(n // BR,),
        in_specs=[pl.BlockSpec((2, BR, D), lambda i: (0, i, 0)),
                  pl.BlockSpec((BR, 1), lambda i: (i, 0)),
                  pl.BlockSpec((D, 2 * D), lambda i: (0, 0)),
                  pl.BlockSpec((1, 2 * D), lambda i: (0, 0))],
        out_specs=pl.BlockSpec((BR, 2 * D), lambda i: (i, 0)),
    )(parts, deg, wb, bb)


def _att_raw_kernel(g_ref, w2_ref, b2_ref, o_ref):
    x = g_ref[...]
    l = jnp.where(x > 0, x, 0.01 * x)
    o_ref[...] = jnp.sum(l * w2_ref[...], axis=1, keepdims=True) + b2_ref[0, 0]


def _exp_kernel(x_ref, o_ref):
    x = x_ref[...]
    o_ref[...] = jnp.exp(x - jnp.max(x))


def _div_kernel(p_ref, s_ref, o_ref):
    num = p_ref[0] + p_ref[1]
    den = s_ref[0, :, 0] + s_ref[1, :, 0] + 1e-20
    o_ref[...] = num / den[:, None]


def _bn_tanh_parts_kernel(p_ref, g_ref, b_ref, o_ref):
    x = p_ref[0] + p_ref[1]
    mu = jnp.mean(x, axis=0, keepdims=True)
    var = jnp.mean((x - mu) ** 2, axis=0, keepdims=True)
    o_ref[...] = jnp.tanh((x - mu) / jnp.sqrt(var + 1e-5) * g_ref[...]
                          + b_ref[...])


def _div_bn_tanh_kernel(p_ref, s_ref, g_ref, b_ref, o_ref):
    num = p_ref[0] + p_ref[1]
    den = s_ref[0, :, 0] + s_ref[1, :, 0] + 1e-20
    x = num / den[:, None]
    mu = jnp.mean(x, axis=0, keepdims=True)
    var = jnp.mean((x - mu) ** 2, axis=0, keepdims=True)
    o_ref[...] = jnp.tanh((x - mu) / jnp.sqrt(var + 1e-5) * g_ref[...]
                          + b_ref[...])


def _gru_kernel(h_ref, hv_ref,
                wiz_ref, biz_ref, whz_ref, bhz_ref,
                wir_ref, bir_ref, whr_ref, bhr_ref,
                win_ref, bin_ref, whn_ref, bhn_ref, o_ref):
    h = h_ref[...]
    hv = hv_ref[...]

    def mm(x, w_ref, b_ref):
        return jnp.dot(x, w_ref[...], preferred_element_type=jnp.float32,
                       precision=lax.Precision.HIGHEST) + b_ref[...]

    z = jax.nn.sigmoid(mm(h, wiz_ref, biz_ref) + mm(hv, whz_ref, bhz_ref))
    r = jax.nn.sigmoid(mm(h, wir_ref, bir_ref) + mm(hv, whr_ref, bhr_ref))
    n = jnp.tanh(mm(h, win_ref, bin_ref) + mm(r * hv, whn_ref, bhn_ref))
    o_ref[...] = (1.0 - z) * n + z * hv


def _gru(h, hv, wb):
    n = h.shape[0]
    BR = 1000
    ws = pl.BlockSpec((D, D), lambda i: (0, 0))
    bs = pl.BlockSpec((1, D), lambda i: (0, 0))
    xs = pl.BlockSpec((BR, D), lambda i: (i, 0))
    return pl.pallas_call(
        _gru_kernel,
        out_shape=jax.ShapeDtypeStruct((n, D), jnp.float32),
        grid=(n // BR,),
        in_specs=[xs, xs] + [ws, bs] * 6,
        out_specs=xs,
    )(h, hv, *wb)


def _att_raw(G, W2t, b2):
    nnz = G.shape[0]
    BR = 2560
    grid = (nnz // BR,)
    out = pl.pallas_call(
        _att_raw_kernel,
        out_shape=jax.ShapeDtypeStruct((nnz, 1), jnp.float32),
        grid=grid,
        in_specs=[pl.BlockSpec((BR, G.shape[1]), lambda i: (i, 0)),
                  pl.BlockSpec((1, G.shape[1]), lambda i: (0, 0)),
                  pl.BlockSpec((1, 1), lambda i: (0, 0),
                               memory_space=pltpu.SMEM)],
        out_specs=pl.BlockSpec((BR, 1), lambda i: (i, 0)),
    )(G, W2t, b2)
    return out


def _pad_rows(n, mult=2048):
    return -(-n // mult) * mult


def kernel(h, incident_nodes, incident_edges, incident_values,
           degree_v_values, degree_e_values, sent_index, sent_values,
           layer, params):
    p = params
    N_v = h.shape[0]
    N_e = degree_e_values.shape[0]
    nv_pad = _pad_rows(N_v)
    ne_pad = _pad_rows(N_e)
    nnz = incident_nodes.shape[0]

    P_ve, W_ve, nch_ve = _pack3(incident_nodes, incident_edges, incident_values)
    P_ev, W_ev, nch_ev = _pack3(incident_edges, incident_nodes, incident_values)
    P_sent, W_sent, nch_sent = _pack3(sent_index[1], sent_index[0], sent_values)
    PG1, nchg1 = _pack2(incident_nodes, incident_edges)
    PG2, nchg2 = _pack2(incident_edges, incident_nodes)

    r1 = lambda x: x.reshape(1, -1)

    h_m, A1 = _mlp1(h, p['mlp1_W1'], r1(p['mlp1_b1']),
                    p['mlp1_W2'], r1(p['mlp1_b2']), p['att1_W1'][:D])

    ht_parts = _sc_spmm(h_m, P_ve, W_ve, nch_ve, ne_pad)

    B1 = _finalize_b(ht_parts[:, :N_e], degree_e_values.reshape(N_e, 1),
                     p['att1_W1'][D:], r1(p['att1_b1']))

    G = _sc_gather_add(A1, B1, PG1, nchg1)

    raw = _att_raw(G[:nnz], p['att1_W2'].reshape(1, 2 * D),
                   p['att1_b2'].reshape(1, 1))
    att = _tc(_exp_kernel, jax.ShapeDtypeStruct((nnz // D, D), jnp.float32),
              raw.reshape(nnz // D, D)).reshape(nnz)

    P_att, W_att, nch_att = _pack3(incident_nodes, incident_edges, att)
    pooled_parts = _sc_spmm(h_m, P_att, W_att, nch_att, ne_pad)
    rs_parts = _sc_rowsum(P_att, W_att, nch_att, ne_pad)

    h_e = _tc(_div_kernel, jax.ShapeDtypeStruct((N_e, D), jnp.float32),
              pooled_parts[:, :N_e], rs_parts[:, :N_e])

    hn_parts = _sc_spmm(h_e, P_sent, W_sent, nch_sent, ne_pad)

    hn_bn = _tc(_bn_tanh_parts_kernel,
                jax.ShapeDtypeStruct((N_e, D), jnp.float32),
                hn_parts[:, :N_e], r1(p['bn2_g']), r1(p['bn2_b']))
    h_n, A2 = _mlp1(hn_bn, p['mlp2_W1'], r1(p['mlp2_b1']),
                    p['mlp2_W2'], r1(p['mlp2_b2']), p['att2_W1'][:D])

    ht2_parts = _sc_spmm(h_n, P_ev, W_ev, nch_ev, nv_pad)

    B2 = _finalize_b(ht2_parts[:, :N_v], degree_v_values.reshape(N_v, 1),
                     p['att2_W1'][D:], r1(p['att2_b1']))

    G2 = _sc_gather_add(A2, B2, PG2, nchg2)

    raw2 = _att_raw(G2[:nnz], p['att2_W2'].reshape(1, 2 * D),
                    p['att2_b2'].reshape(1, 1))
    att2 = _tc(_exp_kernel, jax.ShapeDtypeStruct((nnz // D, D), jnp.float32),
               raw2.reshape(nnz // D, D)).reshape(nnz)

    P_att2, W_att2, nch_att2 = _pack3(incident_edges, incident_nodes, att2)
    pooled2_parts = _sc_spmm(h_n, P_att2, W_att2, nch_att2, nv_pad)
    rs2_parts = _sc_rowsum(P_att2, W_att2, nch_att2, nv_pad)

    h_v = _tc(_div_bn_tanh_kernel,
              jax.ShapeDtypeStruct((N_v, D), jnp.float32),
              pooled2_parts[:, :N_v], rs2_parts[:, :N_v],
              r1(p['bn1_g']), r1(p['bn1_b']))
    out = _gru(h, h_v,
               (p['gru_Wiz'], r1(p['gru_biz']), p['gru_Whz'], r1(p['gru_bhz']),
                p['gru_Wir'], r1(p['gru_bir']), p['gru_Whr'], r1(p['gru_bhr']),
                p['gru_Win'], r1(p['gru_bin']), p['gru_Whn'], r1(p['gru_bhn'])))
    return out


# speedup vs baseline: 1.6396x; 1.1432x over previous
import functools

import jax
import jax.numpy as jnp
from jax import lax
from jax.experimental import pallas as pl
from jax.experimental.pallas import tpu as pltpu
from jax.experimental.pallas import tpu_sc as plsc

NW = 32
CHUNK = 128
D = 128


def _pack3(idx_src, idx_dst, w, C=CHUNK):
    nnz = idx_src.shape[0]
    nch = -(-nnz // (C * NW))
    nch += nch % 2
    tot = nch * NW * C
    pad = tot - nnz
    a = jnp.pad(idx_src.astype(jnp.int32), (0, pad))
    b = jnp.pad(idx_dst.astype(jnp.int32), (0, pad))
    P = jnp.stack([a, b]).reshape(2, nch * NW, C).transpose(1, 0, 2)
    W = jnp.pad(w, (0, pad)).reshape(nch * NW, 1, C)
    return P, W, nch


def _pack2(idx_a, idx_b, C):
    nnz = idx_a.shape[0]
    nch = -(-nnz // (C * NW))
    nch += nch % 2
    tot = nch * NW * C
    pad = tot - nnz
    a = jnp.pad(idx_a.astype(jnp.int32), (0, pad))
    b = jnp.pad(idx_b.astype(jnp.int32), (0, pad))
    P = jnp.stack([a, b]).reshape(2, nch * NW, C).transpose(1, 0, 2)
    return P, nch


def _bcast_lane(v, j):
    return lax.gather(
        v, jnp.full((16, 1), j, jnp.int32),
        lax.GatherDimensionNumbers(offset_dims=(), collapsed_slice_dims=(0,),
                                   start_index_map=(0,)),
        (1,), mode=lax.GatherScatterMode.PROMISE_IN_BOUNDS)

def _sc_spmm(table, P, W, nch, np_pad):
    TW = table.shape[1]
    out_type = jax.ShapeDtypeStruct((2, np_pad, TW), jnp.float32)
    scratch = [
        pltpu.VMEM((2, CHUNK), jnp.int32), pltpu.VMEM((2, CHUNK), jnp.int32),
        pltpu.VMEM((1, CHUNK), jnp.float32), pltpu.VMEM((1, CHUNK), jnp.float32),
        pltpu.VMEM((CHUNK, TW), jnp.float32), pltpu.VMEM((CHUNK, TW), jnp.float32),
        pltpu.VMEM_SHARED((np_pad, TW), jnp.float32),
        pltpu.SemaphoreType.DMA, pltpu.SemaphoreType.DMA,
    ]
    mesh = plsc.VectorSubcoreMesh(core_axis_name="c", subcore_axis_name="s")

    def body(table_h, p_h, w_h, outp_h, pbuf0, pbuf1, wbuf0, wbuf1,
             rows0, rows1, accp, gsem0, gsem1):
        pbuf = (pbuf0, pbuf1)
        wbuf = (wbuf0, wbuf1)
        rows = (rows0, rows1)
        gsem = (gsem0, gsem1)
        cid = lax.axis_index("c")
        sid = lax.axis_index("s")
        z16 = jnp.zeros((16,), jnp.float32)

        def zb(i, _):
            for r in range(TW // 16):
                rows0[i, pl.ds(r * 16, 16)] = z16
            return 0
        lax.fori_loop(0, 128, zb, 0)

        rpt = np_pad // 16
        for s2 in range(rpt // 128):
            r0 = sid * rpt + s2 * 128
            pltpu.sync_copy(rows0, accp.at[pl.ds(r0, 128)])
        plsc.subcore_barrier()

        base0 = (cid * 16 + sid) * nch

        def prefetch(ch, t):
            pltpu.sync_copy(p_h.at[ch], pbuf[t])
            pltpu.sync_copy(w_h.at[ch], wbuf[t])
            pltpu.async_copy(table_h.at[pbuf[t].at[0]], rows[t], gsem[t])

        prefetch(base0, 0)

        def pair(i2, _):
            base = base0 + 2 * i2
            for b in range(2):
                ch = base + b
                t = b
                if b == 0:
                    prefetch(ch + 1, 1)
                else:
                    @pl.when(i2 < nch // 2 - 1)
                    def _():
                        prefetch(ch + 1, 0)
                pltpu.make_async_copy(table_h.at[pbuf[t].at[0]], rows[t],
                                      gsem[t]).wait()

                def scale(j16, _):
                    w16 = wbuf[t][0, pl.ds(j16 * 16, 16)]
                    for j in range(16):
                        wv = _bcast_lane(w16, j)
                        row = j16 * 16 + j
                        for r in range(TW // 16):
                            rows[t][row, pl.ds(r * 16, 16)] = (
                                rows[t][row, pl.ds(r * 16, 16)] * wv)
                    return 0
                lax.fori_loop(0, CHUNK // 16, scale, 0)
                pltpu.sync_copy(rows[t], accp.at[pbuf[t].at[1]], add=True)
            return 0
        lax.fori_loop(0, nch // 2, pair, 0)
        plsc.subcore_barrier()

        for s2 in range(rpt // 128):
            r0 = sid * rpt + s2 * 128
            pltpu.sync_copy(accp.at[pl.ds(r0, 128)], outp_h.at[cid, pl.ds(r0, 128)])

    k = pl.kernel(body, out_type=out_type, mesh=mesh, scratch_types=scratch)
    return k(table, P, W)


def _sc_rowsum(P, W, nch, np_pad):
    out_type = jax.ShapeDtypeStruct((2, np_pad, D), jnp.float32)
    scratch = [
        pltpu.VMEM((2, CHUNK), jnp.int32),
        pltpu.VMEM((1, CHUNK), jnp.float32),
        pltpu.VMEM((CHUNK, D), jnp.float32),
        pltpu.VMEM_SHARED((np_pad, D), jnp.float32),
        pltpu.SemaphoreType.DMA,
    ]
    mesh = plsc.VectorSubcoreMesh(core_axis_name="c", subcore_axis_name="s")

    def body(p_h, w_h, outp_h, pbuf, wbuf, rows, accp, sem):
        cid = lax.axis_index("c")
        sid = lax.axis_index("s")
        z16 = jnp.zeros((16,), jnp.float32)
        i16 = lax.iota(jnp.int32, 16)

        def zb(i, _):
            for r in range(D // 16):
                rows[i, pl.ds(r * 16, 16)] = z16
            return 0
        lax.fori_loop(0, 128, zb, 0)

        rpt = np_pad // 16
        for s2 in range(rpt // 128):
            r0 = sid * rpt + s2 * 128
            pltpu.sync_copy(rows, accp.at[pl.ds(r0, 128)])
        plsc.subcore_barrier()

        def chunk(i, _):
            ch = (cid * 16 + sid) * nch + i
            pltpu.sync_copy(p_h.at[ch], pbuf)
            pltpu.sync_copy(w_h.at[ch], wbuf)

            def scale(j16, _):
                w16 = wbuf[0, pl.ds(j16 * 16, 16)]
                for j in range(16):
                    wv = _bcast_lane(w16, j)
                    rows[j16 * 16 + j, pl.ds(0, 16)] = jnp.where(i16 == 0, wv, 0.0)
                return 0
            lax.fori_loop(0, CHUNK // 16, scale, 0)
            pltpu.sync_copy(rows, accp.at[pbuf.at[1]], add=True)
            return 0
        lax.fori_loop(0, nch, chunk, 0)
        plsc.subcore_barrier()

        for s2 in range(rpt // 128):
            r0 = sid * rpt + s2 * 128
            pltpu.sync_copy(accp.at[pl.ds(r0, 128)], outp_h.at[cid, pl.ds(r0, 128)])

    k = pl.kernel(body, out_type=out_type, mesh=mesh, scratch_types=scratch)
    return k(P, W)


CHUNK_G = 64


def _sc_gather_add(A, B, P, nch):
    tot = P.shape[0] * CHUNK_G
    W = A.shape[1]
    scratch = [
        pltpu.VMEM((2, CHUNK_G), jnp.int32), pltpu.VMEM((2, CHUNK_G), jnp.int32),
        pltpu.VMEM((CHUNK_G, W), jnp.float32), pltpu.VMEM((CHUNK_G, W), jnp.float32),
        pltpu.VMEM((CHUNK_G, W), jnp.float32), pltpu.VMEM((CHUNK_G, W), jnp.float32),
        pltpu.SemaphoreType.DMA, pltpu.SemaphoreType.DMA,
        pltpu.SemaphoreType.DMA, pltpu.SemaphoreType.DMA,
        pltpu.SemaphoreType.DMA, pltpu.SemaphoreType.DMA,
    ]
    mesh = plsc.VectorSubcoreMesh(core_axis_name="c", subcore_axis_name="s")

    def body(a_h, b_h, p_h, g_h, pbuf0, pbuf1, ga0, ga1, gb0, gb1,
             sa0, sa1, sb0, sb1, sw0, sw1):
        pbuf = (pbuf0, pbuf1)
        ga = (ga0, ga1)
        gb = (gb0, gb1)
        sa = (sa0, sa1)
        sb = (sb0, sb1)
        sw = (sw0, sw1)
        cid = lax.axis_index("c")
        sid = lax.axis_index("s")
        base0 = (cid * 16 + sid) * nch

        def prefetch(ch, t):
            pltpu.sync_copy(p_h.at[ch], pbuf[t])
            pltpu.async_copy(a_h.at[pbuf[t].at[0]], ga[t], sa[t])
            pltpu.async_copy(b_h.at[pbuf[t].at[1]], gb[t], sb[t])

        prefetch(base0, 0)

        def pair(i2, _):
            base = base0 + 2 * i2
            for b in range(2):
                ch = base + b
                t = b
                if b == 0:
                    prefetch(ch + 1, 1)
                else:
                    @pl.when(i2 < nch // 2 - 1)
                    def _():
                        prefetch(ch + 1, 0)
                pltpu.make_async_copy(a_h.at[pbuf[t].at[0]], ga[t], sa[t]).wait()
                pltpu.make_async_copy(b_h.at[pbuf[t].at[1]], gb[t], sb[t]).wait()

                @pl.when(i2 >= 1)
                def _():
                    pltpu.make_async_copy(
                        ga[t], g_h.at[pl.ds(0, CHUNK_G)], sw[t]).wait()

                def add(j, _):
                    for r in range(W // 16):
                        ga[t][j, pl.ds(r * 16, 16)] = (
                            ga[t][j, pl.ds(r * 16, 16)]
                            + gb[t][j, pl.ds(r * 16, 16)])
                    return 0
                lax.fori_loop(0, CHUNK_G, add, 0)
                pltpu.async_copy(ga[t], g_h.at[pl.ds(ch * CHUNK_G, CHUNK_G)],
                                 sw[t])
            return 0
        lax.fori_loop(0, nch // 2, pair, 0)
        for t in range(2):
            pltpu.make_async_copy(ga[t], g_h.at[pl.ds(0, CHUNK_G)],
                                  sw[t]).wait()

    k = pl.kernel(body, out_type=jax.ShapeDtypeStruct((tot, W), jnp.float32),
                  mesh=mesh, scratch_types=scratch)
    return k(A, B, P)


def _tc(fn, out_shape, *args):
    return pl.pallas_call(fn, out_shape=out_shape)(*args)


def _mlp1_kernel(h_ref, w1_ref, b1_ref, w2_ref, b2_ref, wa_ref,
                 hm_ref, a1_ref):
    h = h_ref[...]
    t = jnp.maximum(jnp.dot(h, w1_ref[...], preferred_element_type=jnp.float32,
                            precision=lax.Precision.HIGHEST) + b1_ref[...], 0.0)
    hm = jnp.dot(t, w2_ref[...], preferred_element_type=jnp.float32,
                 precision=lax.Precision.HIGHEST) + b2_ref[...]
    hm_ref[...] = hm
    a1_ref[...] = jnp.dot(hm, wa_ref[...], preferred_element_type=jnp.float32,
                          precision=lax.Precision.HIGHEST)


def _mlp1(h, w1, b1, w2, b2, wa):
    n = h.shape[0]
    BR = 1000
    return pl.pallas_call(
        _mlp1_kernel,
        out_shape=(jax.ShapeDtypeStruct((n, D), jnp.float32),
                   jax.ShapeDtypeStruct((n, 2 * D), jnp.float32)),
        grid=(n // BR,),
        in_specs=[pl.BlockSpec((BR, D), lambda i: (i, 0)),
                  pl.BlockSpec((D, D), lambda i: (0, 0)),
                  pl.BlockSpec((1, D), lambda i: (0, 0)),
                  pl.BlockSpec((D, D), lambda i: (0, 0)),
                  pl.BlockSpec((1, D), lambda i: (0, 0)),
                  pl.BlockSpec((D, 2 * D), lambda i: (0, 0))],
        out_specs=(pl.BlockSpec((BR, D), lambda i: (i, 0)),
                   pl.BlockSpec((BR, 2 * D), lambda i: (i, 0))),
    )(h, w1, b1, w2, b2, wa)


def _finalize_b_kernel(p_ref, deg_ref, wb_ref, bb_ref, b_ref):
    ht = (p_ref[0] + p_ref[1]) * deg_ref[...]
    b_ref[...] = jnp.dot(ht, wb_ref[...],
                         preferred_element_type=jnp.float32,
                         precision=lax.Precision.HIGHEST) + bb_ref[...]


def _finalize_b(parts, deg, wb, bb):
    n = parts.shape[1]
    BR = 1000
    return pl.pallas_call(
        _finalize_b_kernel,
        out_shape=jax.ShapeDtypeStruct((n, 2 * D), jnp.float32),
        grid=(n // BR,),
        in_specs=[pl.BlockSpec((2, BR, D), lambda i: (0, i, 0)),
                  pl.BlockSpec((BR, 1), lambda i: (i, 0)),
                  pl.BlockSpec((D, 2 * D), lambda i: (0, 0)),
                  pl.BlockSpec((1, 2 * D), lambda i: (0, 0))],
        out_specs=pl.BlockSpec((BR, 2 * D), lambda i: (i, 0)),
    )(parts, deg, wb, bb)


def _att_raw_kernel(g_ref, w2_ref, b2_ref, o_ref):
    x = g_ref[...]
    l = jnp.where(x > 0, x, 0.01 * x)
    o_ref[...] = jnp.sum(l * w2_ref[...], axis=1, keepdims=True) + b2_ref[0, 0]


def _exp_kernel(x_ref, o_ref):
    x = x_ref[...]
    o_ref[...] = jnp.exp(x - jnp.max(x))


def _div_kernel(p_ref, s_ref, o_ref):
    num = p_ref[0] + p_ref[1]
    den = s_ref[0, :, 0] + s_ref[1, :, 0] + 1e-20
    o_ref[...] = num / den[:, None]


def _bn_tanh_parts_kernel(p_ref, g_ref, b_ref, o_ref):
    x = p_ref[0] + p_ref[1]
    mu = jnp.mean(x, axis=0, keepdims=True)
    var = jnp.mean((x - mu) ** 2, axis=0, keepdims=True)
    o_ref[...] = jnp.tanh((x - mu) / jnp.sqrt(var + 1e-5) * g_ref[...]
                          + b_ref[...])


def _div_bn_tanh_kernel(p_ref, s_ref, g_ref, b_ref, o_ref):
    num = p_ref[0] + p_ref[1]
    den = s_ref[0, :, 0] + s_ref[1, :, 0] + 1e-20
    x = num / den[:, None]
    mu = jnp.mean(x, axis=0, keepdims=True)
    var = jnp.mean((x - mu) ** 2, axis=0, keepdims=True)
    o_ref[...] = jnp.tanh((x - mu) / jnp.sqrt(var + 1e-5) * g_ref[...]
                          + b_ref[...])


def _gru_kernel(h_ref, hv_ref,
                wiz_ref, biz_ref, whz_ref, bhz_ref,
                wir_ref, bir_ref, whr_ref, bhr_ref,
                win_ref, bin_ref, whn_ref, bhn_ref, o_ref):
    h = h_ref[...]
    hv = hv_ref[...]

    def mm(x, w_ref, b_ref):
        return jnp.dot(x, w_ref[...], preferred_element_type=jnp.float32,
                       precision=lax.Precision.HIGHEST) + b_ref[...]

    z = jax.nn.sigmoid(mm(h, wiz_ref, biz_ref) + mm(hv, whz_ref, bhz_ref))
    r = jax.nn.sigmoid(mm(h, wir_ref, bir_ref) + mm(hv, whr_ref, bhr_ref))
    n = jnp.tanh(mm(h, win_ref, bin_ref) + mm(r * hv, whn_ref, bhn_ref))
    o_ref[...] = (1.0 - z) * n + z * hv


def _gru(h, hv, wb):
    n = h.shape[0]
    BR = 1000
    ws = pl.BlockSpec((D, D), lambda i: (0, 0))
    bs = pl.BlockSpec((1, D), lambda i: (0, 0))
    xs = pl.BlockSpec((BR, D), lambda i: (i, 0))
    return pl.pallas_call(
        _gru_kernel,
        out_shape=jax.ShapeDtypeStruct((n, D), jnp.float32),
        grid=(n // BR,),
        in_specs=[xs, xs] + [ws, bs] * 6,
        out_specs=xs,
    )(h, hv, *wb)


def _att_raw(G, W2t, b2):
    nnz = G.shape[0]
    BR = 2560
    grid = (nnz // BR,)
    out = pl.pallas_call(
        _att_raw_kernel,
        out_shape=jax.ShapeDtypeStruct((nnz, 1), jnp.float32),
        grid=grid,
        in_specs=[pl.BlockSpec((BR, G.shape[1]), lambda i: (i, 0)),
                  pl.BlockSpec((1, G.shape[1]), lambda i: (0, 0)),
                  pl.BlockSpec((1, 1), lambda i: (0, 0),
                               memory_space=pltpu.SMEM)],
        out_specs=pl.BlockSpec((BR, 1), lambda i: (i, 0)),
    )(G, W2t, b2)
    return out


def _pad_rows(n, mult=2048):
    return -(-n // mult) * mult


def kernel(h, incident_nodes, incident_edges, incident_values,
           degree_v_values, degree_e_values, sent_index, sent_values,
           layer, params):
    p = params
    N_v = h.shape[0]
    N_e = degree_e_values.shape[0]
    nv_pad = _pad_rows(N_v)
    ne_pad = _pad_rows(N_e)
    nnz = incident_nodes.shape[0]

    P_ve, W_ve, nch_ve = _pack3(incident_nodes, incident_edges, incident_values)
    P_ev, W_ev, nch_ev = _pack3(incident_edges, incident_nodes, incident_values)
    P_sent, W_sent, nch_sent = _pack3(sent_index[1], sent_index[0], sent_values)
    PG1, nchg1 = _pack2(incident_nodes, incident_edges, CHUNK_G)
    PG2, nchg2 = _pack2(incident_edges, incident_nodes, CHUNK_G)

    r1 = lambda x: x.reshape(1, -1)

    h_m, A1 = _mlp1(h, p['mlp1_W1'], r1(p['mlp1_b1']),
                    p['mlp1_W2'], r1(p['mlp1_b2']), p['att1_W1'][:D])

    ht_parts = _sc_spmm(h_m, P_ve, W_ve, nch_ve, ne_pad)

    B1 = _finalize_b(ht_parts[:, :N_e], degree_e_values.reshape(N_e, 1),
                     p['att1_W1'][D:], r1(p['att1_b1']))

    G = _sc_gather_add(A1, B1, PG1, nchg1)

    raw = _att_raw(G[:nnz], p['att1_W2'].reshape(1, 2 * D),
                   p['att1_b2'].reshape(1, 1))
    att = _tc(_exp_kernel, jax.ShapeDtypeStruct((nnz // D, D), jnp.float32),
              raw.reshape(nnz // D, D)).reshape(nnz)

    P_att, W_att, nch_att = _pack3(incident_nodes, incident_edges, att)
    pooled_parts = _sc_spmm(h_m, P_att, W_att, nch_att, ne_pad)
    rs_parts = _sc_rowsum(P_att, W_att, nch_att, ne_pad)

    h_e = _tc(_div_kernel, jax.ShapeDtypeStruct((N_e, D), jnp.float32),
              pooled_parts[:, :N_e], rs_parts[:, :N_e])

    hn_parts = _sc_spmm(h_e, P_sent, W_sent, nch_sent, ne_pad)

    hn_bn = _tc(_bn_tanh_parts_kernel,
                jax.ShapeDtypeStruct((N_e, D), jnp.float32),
                hn_parts[:, :N_e], r1(p['bn2_g']), r1(p['bn2_b']))
    h_n, A2 = _mlp1(hn_bn, p['mlp2_W1'], r1(p['mlp2_b1']),
                    p['mlp2_W2'], r1(p['mlp2_b2']), p['att2_W1'][:D])

    ht2_parts = _sc_spmm(h_n, P_ev, W_ev, nch_ev, nv_pad)

    B2 = _finalize_b(ht2_parts[:, :N_v], degree_v_values.reshape(N_v, 1),
                     p['att2_W1'][D:], r1(p['att2_b1']))

    G2 = _sc_gather_add(A2, B2, PG2, nchg2)

    raw2 = _att_raw(G2[:nnz], p['att2_W2'].reshape(1, 2 * D),
                    p['att2_b2'].reshape(1, 1))
    att2 = _tc(_exp_kernel, jax.ShapeDtypeStruct((nnz // D, D), jnp.float32),
               raw2.reshape(nnz // D, D)).reshape(nnz)

    P_att2, W_att2, nch_att2 = _pack3(incident_edges, incident_nodes, att2)
    pooled2_parts = _sc_spmm(h_n, P_att2, W_att2, nch_att2, nv_pad)
    rs2_parts = _sc_rowsum(P_att2, W_att2, nch_att2, nv_pad)

    h_v = _tc(_div_bn_tanh_kernel,
              jax.ShapeDtypeStruct((N_v, D), jnp.float32),
              pooled2_parts[:, :N_v], rs2_parts[:, :N_v],
              r1(p['bn1_g']), r1(p['bn1_b']))
    out = _gru(h, h_v,
               (p['gru_Wiz'], r1(p['gru_biz']), p['gru_Whz'], r1(p['gru_bhz']),
                p['gru_Wir'], r1(p['gru_bir']), p['gru_Whr'], r1(p['gru_bhr']),
                p['gru_Win'], r1(p['gru_bin']), p['gru_Whn'], r1(p['gru_bhn'])))
    return out


# speedup vs baseline: 1.6848x; 1.0276x over previous
import functools

import jax
import jax.numpy as jnp
from jax import lax
from jax.experimental import pallas as pl
from jax.experimental.pallas import tpu as pltpu
from jax.experimental.pallas import tpu_sc as plsc

NW = 32
CHUNK = 128
D = 128


def _pack3(idx_src, idx_dst, w, C=CHUNK):
    nnz = idx_src.shape[0]
    nch = -(-nnz // (C * NW))
    nch += nch % 2
    tot = nch * NW * C
    pad = tot - nnz
    a = jnp.pad(idx_src.astype(jnp.int32), (0, pad))
    b = jnp.pad(idx_dst.astype(jnp.int32), (0, pad))
    P = jnp.stack([a, b]).reshape(2, nch * NW, C).transpose(1, 0, 2)
    W = jnp.pad(w, (0, pad)).reshape(nch * NW, 1, C)
    return P, W, nch


def _pack2(idx_a, idx_b, C):
    nnz = idx_a.shape[0]
    nch = -(-nnz // (C * NW))
    nch += nch % 2
    tot = nch * NW * C
    pad = tot - nnz
    a = jnp.pad(idx_a.astype(jnp.int32), (0, pad))
    b = jnp.pad(idx_b.astype(jnp.int32), (0, pad))
    P = jnp.stack([a, b]).reshape(2, nch * NW, C).transpose(1, 0, 2)
    return P, nch


def _bcast_lane(v, j):
    return lax.gather(
        v, jnp.full((16, 1), j, jnp.int32),
        lax.GatherDimensionNumbers(offset_dims=(), collapsed_slice_dims=(0,),
                                   start_index_map=(0,)),
        (1,), mode=lax.GatherScatterMode.PROMISE_IN_BOUNDS)

def _sc_spmm(table, P, W, nch, np_pad):
    TW = table.shape[1]
    out_type = jax.ShapeDtypeStruct((2, np_pad, TW), jnp.float32)
    scratch = [
        pltpu.VMEM((2, CHUNK), jnp.int32), pltpu.VMEM((2, CHUNK), jnp.int32),
        pltpu.VMEM((1, CHUNK), jnp.float32), pltpu.VMEM((1, CHUNK), jnp.float32),
        pltpu.VMEM((CHUNK, TW), jnp.float32), pltpu.VMEM((CHUNK, TW), jnp.float32),
        pltpu.VMEM_SHARED((np_pad, TW), jnp.float32),
        pltpu.SemaphoreType.DMA, pltpu.SemaphoreType.DMA,
    ]
    mesh = plsc.VectorSubcoreMesh(core_axis_name="c", subcore_axis_name="s")

    def body(table_h, p_h, w_h, outp_h, pbuf0, pbuf1, wbuf0, wbuf1,
             rows0, rows1, accp, gsem0, gsem1):
        pbuf = (pbuf0, pbuf1)
        wbuf = (wbuf0, wbuf1)
        rows = (rows0, rows1)
        gsem = (gsem0, gsem1)
        cid = lax.axis_index("c")
        sid = lax.axis_index("s")
        z16 = jnp.zeros((16,), jnp.float32)

        def zb(i, _):
            for r in range(TW // 16):
                rows0[i, pl.ds(r * 16, 16)] = z16
            return 0
        lax.fori_loop(0, 128, zb, 0)

        rpt = np_pad // 16
        for s2 in range(rpt // 128):
            r0 = sid * rpt + s2 * 128
            pltpu.sync_copy(rows0, accp.at[pl.ds(r0, 128)])
        plsc.subcore_barrier()

        base0 = (cid * 16 + sid) * nch

        def prefetch(ch, t):
            pltpu.sync_copy(p_h.at[ch], pbuf[t])
            pltpu.sync_copy(w_h.at[ch], wbuf[t])
            pltpu.async_copy(table_h.at[pbuf[t].at[0]], rows[t], gsem[t])

        prefetch(base0, 0)

        def pair(i2, _):
            base = base0 + 2 * i2
            for b in range(2):
                ch = base + b
                t = b
                if b == 0:
                    prefetch(ch + 1, 1)
                else:
                    @pl.when(i2 < nch // 2 - 1)
                    def _():
                        prefetch(ch + 1, 0)
                pltpu.make_async_copy(table_h.at[pbuf[t].at[0]], rows[t],
                                      gsem[t]).wait()

                def scale(j16, _):
                    w16 = wbuf[t][0, pl.ds(j16 * 16, 16)]
                    for j in range(16):
                        wv = _bcast_lane(w16, j)
                        row = j16 * 16 + j
                        for r in range(TW // 16):
                            rows[t][row, pl.ds(r * 16, 16)] = (
                                rows[t][row, pl.ds(r * 16, 16)] * wv)
                    return 0
                lax.fori_loop(0, CHUNK // 16, scale, 0)
                pltpu.sync_copy(rows[t], accp.at[pbuf[t].at[1]], add=True)
            return 0
        lax.fori_loop(0, nch // 2, pair, 0)
        plsc.subcore_barrier()

        for s2 in range(rpt // 128):
            r0 = sid * rpt + s2 * 128
            pltpu.sync_copy(accp.at[pl.ds(r0, 128)], outp_h.at[cid, pl.ds(r0, 128)])

    k = pl.kernel(body, out_type=out_type, mesh=mesh, scratch_types=scratch)
    return k(table, P, W)


def _sc_rowsum(P, W, nch, np_pad):
    out_type = jax.ShapeDtypeStruct((2, np_pad, D), jnp.float32)
    scratch = [
        pltpu.VMEM((2, CHUNK), jnp.int32),
        pltpu.VMEM((1, CHUNK), jnp.float32),
        pltpu.VMEM((CHUNK, D), jnp.float32),
        pltpu.VMEM_SHARED((np_pad, D), jnp.float32),
        pltpu.SemaphoreType.DMA,
    ]
    mesh = plsc.VectorSubcoreMesh(core_axis_name="c", subcore_axis_name="s")

    def body(p_h, w_h, outp_h, pbuf, wbuf, rows, accp, sem):
        cid = lax.axis_index("c")
        sid = lax.axis_index("s")
        z16 = jnp.zeros((16,), jnp.float32)
        i16 = lax.iota(jnp.int32, 16)

        def zb(i, _):
            for r in range(D // 16):
                rows[i, pl.ds(r * 16, 16)] = z16
            return 0
        lax.fori_loop(0, 128, zb, 0)

        rpt = np_pad // 16
        for s2 in range(rpt // 128):
            r0 = sid * rpt + s2 * 128
            pltpu.sync_copy(rows, accp.at[pl.ds(r0, 128)])
        plsc.subcore_barrier()

        def chunk(i, _):
            ch = (cid * 16 + sid) * nch + i
            pltpu.sync_copy(p_h.at[ch], pbuf)
            pltpu.sync_copy(w_h.at[ch], wbuf)

            def scale(j16, _):
                w16 = wbuf[0, pl.ds(j16 * 16, 16)]
                for j in range(16):
                    wv = _bcast_lane(w16, j)
                    rows[j16 * 16 + j, pl.ds(0, 16)] = jnp.where(i16 == 0, wv, 0.0)
                return 0
            lax.fori_loop(0, CHUNK // 16, scale, 0)
            pltpu.sync_copy(rows, accp.at[pbuf.at[1]], add=True)
            return 0
        lax.fori_loop(0, nch, chunk, 0)
        plsc.subcore_barrier()

        for s2 in range(rpt // 128):
            r0 = sid * rpt + s2 * 128
            pltpu.sync_copy(accp.at[pl.ds(r0, 128)], outp_h.at[cid, pl.ds(r0, 128)])

    k = pl.kernel(body, out_type=out_type, mesh=mesh, scratch_types=scratch)
    return k(P, W)


CHUNK_G = 64


def _sc_gather_add(A, B, P, nch):
    tot = P.shape[0] * CHUNK_G
    W = A.shape[1]
    scratch = [
        pltpu.VMEM((2, CHUNK_G), jnp.int32), pltpu.VMEM((2, CHUNK_G), jnp.int32),
        pltpu.VMEM((CHUNK_G, W), jnp.float32), pltpu.VMEM((CHUNK_G, W), jnp.float32),
        pltpu.VMEM((CHUNK_G, W), jnp.float32), pltpu.VMEM((CHUNK_G, W), jnp.float32),
        pltpu.SemaphoreType.DMA, pltpu.SemaphoreType.DMA,
        pltpu.SemaphoreType.DMA, pltpu.SemaphoreType.DMA,
        pltpu.SemaphoreType.DMA, pltpu.SemaphoreType.DMA,
    ]
    mesh = plsc.VectorSubcoreMesh(core_axis_name="c", subcore_axis_name="s")

    def body(a_h, b_h, p_h, g_h, pbuf0, pbuf1, ga0, ga1, gb0, gb1,
             sa0, sa1, sb0, sb1, sw0, sw1):
        pbuf = (pbuf0, pbuf1)
        ga = (ga0, ga1)
        gb = (gb0, gb1)
        sa = (sa0, sa1)
        sb = (sb0, sb1)
        sw = (sw0, sw1)
        cid = lax.axis_index("c")
        sid = lax.axis_index("s")
        base0 = (cid * 16 + sid) * nch

        def prefetch(ch, t):
            pltpu.sync_copy(p_h.at[ch], pbuf[t])
            pltpu.async_copy(a_h.at[pbuf[t].at[0]], ga[t], sa[t])
            pltpu.async_copy(b_h.at[pbuf[t].at[1]], gb[t], sb[t])

        prefetch(base0, 0)

        def pair(i2, _):
            base = base0 + 2 * i2
            for b in range(2):
                ch = base + b
                t = b
                if b == 0:
                    prefetch(ch + 1, 1)
                else:
                    @pl.when(i2 < nch // 2 - 1)
                    def _():
                        prefetch(ch + 1, 0)
                pltpu.make_async_copy(a_h.at[pbuf[t].at[0]], ga[t], sa[t]).wait()
                pltpu.make_async_copy(b_h.at[pbuf[t].at[1]], gb[t], sb[t]).wait()

                @pl.when(i2 >= 1)
                def _():
                    pltpu.make_async_copy(
                        ga[t], g_h.at[pl.ds(0, CHUNK_G)], sw[t]).wait()

                def add(j, _):
                    for r in range(W // 16):
                        ga[t][j, pl.ds(r * 16, 16)] = (
                            ga[t][j, pl.ds(r * 16, 16)]
                            + gb[t][j, pl.ds(r * 16, 16)])
                    return 0
                lax.fori_loop(0, CHUNK_G, add, 0)
                pltpu.async_copy(ga[t], g_h.at[pl.ds(ch * CHUNK_G, CHUNK_G)],
                                 sw[t])
            return 0
        lax.fori_loop(0, nch // 2, pair, 0)
        for t in range(2):
            pltpu.make_async_copy(ga[t], g_h.at[pl.ds(0, CHUNK_G)],
                                  sw[t]).wait()

    k = pl.kernel(body, out_type=jax.ShapeDtypeStruct((tot, W), jnp.float32),
                  mesh=mesh, scratch_types=scratch)
    return k(A, B, P)


def _tc(fn, out_shape, *args):
    return pl.pallas_call(fn, out_shape=out_shape)(*args)


def _bdot(x, w):
    return jnp.dot(x.astype(jnp.bfloat16), w.astype(jnp.bfloat16),
                   preferred_element_type=jnp.float32)


def _mlp1_kernel(h_ref, w1_ref, b1_ref, w2_ref, b2_ref, wa_ref,
                 hm_ref, a1_ref):
    h = h_ref[...]
    t = jnp.maximum(_bdot(h, w1_ref[...]) + b1_ref[...], 0.0)
    hm = _bdot(t, w2_ref[...]) + b2_ref[...]
    hm_ref[...] = hm
    a1_ref[...] = _bdot(hm, wa_ref[...])


def _mlp1(h, w1, b1, w2, b2, wa):
    n = h.shape[0]
    BR = 1000
    return pl.pallas_call(
        _mlp1_kernel,
        out_shape=(jax.ShapeDtypeStruct((n, D), jnp.float32),
                   jax.ShapeDtypeStruct((n, 2 * D), jnp.float32)),
        grid=(n // BR,),
        in_specs=[pl.BlockSpec((BR, D), lambda i: (i, 0)),
                  pl.BlockSpec((D, D), lambda i: (0, 0)),
                  pl.BlockSpec((1, D), lambda i: (0, 0)),
                  pl.BlockSpec((D, D), lambda i: (0, 0)),
                  pl.BlockSpec((1, D), lambda i: (0, 0)),
                  pl.BlockSpec((D, 2 * D), lambda i: (0, 0))],
        out_specs=(pl.BlockSpec((BR, D), lambda i: (i, 0)),
                   pl.BlockSpec((BR, 2 * D), lambda i: (i, 0))),
    )(h, w1, b1, w2, b2, wa)


def _finalize_b_kernel(p_ref, deg_ref, wb_ref, bb_ref, b_ref):
    ht = (p_ref[0] + p_ref[1]) * deg_ref[...]
    b_ref[...] = _bdot(ht, wb_ref[...]) + bb_ref[...]


def _finalize_b(parts, deg, wb, bb):
    n = parts.shape[1]
    BR = 1000
    return pl.pallas_call(
        _finalize_b_kernel,
        out_shape=jax.ShapeDtypeStruct((n, 2 * D), jnp.float32),
        grid=(n // BR,),
        in_specs=[pl.BlockSpec((2, BR, D), lambda i: (0, i, 0)),
                  pl.BlockSpec((BR, 1), lambda i: (i, 0)),
                  pl.BlockSpec((D, 2 * D), lambda i: (0, 0)),
                  pl.BlockSpec((1, 2 * D), lambda i: (0, 0))],
        out_specs=pl.BlockSpec((BR, 2 * D), lambda i: (i, 0)),
    )(parts, deg, wb, bb)


def _att_raw_kernel(g_ref, w2_ref, b2_ref, o_ref):
    x = g_ref[...]
    l = jnp.where(x > 0, x, 0.01 * x).astype(jnp.bfloat16).astype(jnp.float32)
    w = w2_ref[...].astype(jnp.bfloat16).astype(jnp.float32)
    o_ref[...] = jnp.sum(l * w, axis=1, keepdims=True) + b2_ref[0, 0]


def _exp_kernel(x_ref, o_ref):
    x = x_ref[...]
    o_ref[...] = jnp.exp(x - jnp.max(x))


def _div_kernel(p_ref, s_ref, o_ref):
    num = p_ref[0] + p_ref[1]
    den = s_ref[0, :, 0] + s_ref[1, :, 0] + 1e-20
    o_ref[...] = num / den[:, None]


def _bn_tanh_parts_kernel(p_ref, g_ref, b_ref, o_ref):
    x = p_ref[0] + p_ref[1]
    mu = jnp.mean(x, axis=0, keepdims=True)
    var = jnp.mean((x - mu) ** 2, axis=0, keepdims=True)
    o_ref[...] = jnp.tanh((x - mu) / jnp.sqrt(var + 1e-5) * g_ref[...]
                          + b_ref[...])


def _div_bn_tanh_kernel(p_ref, s_ref, g_ref, b_ref, o_ref):
    num = p_ref[0] + p_ref[1]
    den = s_ref[0, :, 0] + s_ref[1, :, 0] + 1e-20
    x = num / den[:, None]
    mu = jnp.mean(x, axis=0, keepdims=True)
    var = jnp.mean((x - mu) ** 2, axis=0, keepdims=True)
    o_ref[...] = jnp.tanh((x - mu) / jnp.sqrt(var + 1e-5) * g_ref[...]
                          + b_ref[...])


def _gru_kernel(h_ref, hv_ref,
                wiz_ref, biz_ref, whz_ref, bhz_ref,
                wir_ref, bir_ref, whr_ref, bhr_ref,
                win_ref, bin_ref, whn_ref, bhn_ref, o_ref):
    h = h_ref[...]
    hv = hv_ref[...]

    def mm(x, w_ref, b_ref):
        return _bdot(x, w_ref[...]) + b_ref[...]

    z = jax.nn.sigmoid(mm(h, wiz_ref, biz_ref) + mm(hv, whz_ref, bhz_ref))
    r = jax.nn.sigmoid(mm(h, wir_ref, bir_ref) + mm(hv, whr_ref, bhr_ref))
    n = jnp.tanh(mm(h, win_ref, bin_ref) + mm(r * hv, whn_ref, bhn_ref))
    o_ref[...] = (1.0 - z) * n + z * hv


def _gru(h, hv, wb):
    n = h.shape[0]
    BR = 1000
    ws = pl.BlockSpec((D, D), lambda i: (0, 0))
    bs = pl.BlockSpec((1, D), lambda i: (0, 0))
    xs = pl.BlockSpec((BR, D), lambda i: (i, 0))
    return pl.pallas_call(
        _gru_kernel,
        out_shape=jax.ShapeDtypeStruct((n, D), jnp.float32),
        grid=(n // BR,),
        in_specs=[xs, xs] + [ws, bs] * 6,
        out_specs=xs,
    )(h, hv, *wb)


def _att_raw(G, W2t, b2):
    nnz = G.shape[0]
    BR = 2560
    grid = (nnz // BR,)
    out = pl.pallas_call(
        _att_raw_kernel,
        out_shape=jax.ShapeDtypeStruct((nnz, 1), jnp.float32),
        grid=grid,
        in_specs=[pl.BlockSpec((BR, G.shape[1]), lambda i: (i, 0)),
                  pl.BlockSpec((1, G.shape[1]), lambda i: (0, 0)),
                  pl.BlockSpec((1, 1), lambda i: (0, 0),
                               memory_space=pltpu.SMEM)],
        out_specs=pl.BlockSpec((BR, 1), lambda i: (i, 0)),
    )(G, W2t, b2)
    return out


def _pad_rows(n, mult=2048):
    return -(-n // mult) * mult


def kernel(h, incident_nodes, incident_edges, incident_values,
           degree_v_values, degree_e_values, sent_index, sent_values,
           layer, params):
    p = params
    N_v = h.shape[0]
    N_e = degree_e_values.shape[0]
    nv_pad = _pad_rows(N_v)
    ne_pad = _pad_rows(N_e)
    nnz = incident_nodes.shape[0]

    P_ve, W_ve, nch_ve = _pack3(incident_nodes, incident_edges, incident_values)
    P_ev, W_ev, nch_ev = _pack3(incident_edges, incident_nodes, incident_values)
    P_sent, W_sent, nch_sent = _pack3(sent_index[1], sent_index[0], sent_values)
    PG1, nchg1 = _pack2(incident_nodes, incident_edges, CHUNK_G)
    PG2, nchg2 = _pack2(incident_edges, incident_nodes, CHUNK_G)

    r1 = lambda x: x.reshape(1, -1)

    h_m, A1 = _mlp1(h, p['mlp1_W1'], r1(p['mlp1_b1']),
                    p['mlp1_W2'], r1(p['mlp1_b2']), p['att1_W1'][:D])

    ht_parts = _sc_spmm(h_m, P_ve, W_ve, nch_ve, ne_pad)

    B1 = _finalize_b(ht_parts[:, :N_e], degree_e_values.reshape(N_e, 1),
                     p['att1_W1'][D:], r1(p['att1_b1']))

    G = _sc_gather_add(A1, B1, PG1, nchg1)

    raw = _att_raw(G[:nnz], p['att1_W2'].reshape(1, 2 * D),
                   p['att1_b2'].reshape(1, 1))
    att = _tc(_exp_kernel, jax.ShapeDtypeStruct((nnz // D, D), jnp.float32),
              raw.reshape(nnz // D, D)).reshape(nnz)

    P_att, W_att, nch_att = _pack3(incident_nodes, incident_edges, att)
    pooled_parts = _sc_spmm(h_m, P_att, W_att, nch_att, ne_pad)
    rs_parts = _sc_rowsum(P_att, W_att, nch_att, ne_pad)

    h_e = _tc(_div_kernel, jax.ShapeDtypeStruct((N_e, D), jnp.float32),
              pooled_parts[:, :N_e], rs_parts[:, :N_e])

    hn_parts = _sc_spmm(h_e, P_sent, W_sent, nch_sent, ne_pad)

    hn_bn = _tc(_bn_tanh_parts_kernel,
                jax.ShapeDtypeStruct((N_e, D), jnp.float32),
                hn_parts[:, :N_e], r1(p['bn2_g']), r1(p['bn2_b']))
    h_n, A2 = _mlp1(hn_bn, p['mlp2_W1'], r1(p['mlp2_b1']),
                    p['mlp2_W2'], r1(p['mlp2_b2']), p['att2_W1'][:D])

    ht2_parts = _sc_spmm(h_n, P_ev, W_ev, nch_ev, nv_pad)

    B2 = _finalize_b(ht2_parts[:, :N_v], degree_v_values.reshape(N_v, 1),
                     p['att2_W1'][D:], r1(p['att2_b1']))

    G2 = _sc_gather_add(A2, B2, PG2, nchg2)

    raw2 = _att_raw(G2[:nnz], p['att2_W2'].reshape(1, 2 * D),
                    p['att2_b2'].reshape(1, 1))
    att2 = _tc(_exp_kernel, jax.ShapeDtypeStruct((nnz // D, D), jnp.float32),
               raw2.reshape(nnz // D, D)).reshape(nnz)

    P_att2, W_att2, nch_att2 = _pack3(incident_edges, incident_nodes, att2)
    pooled2_parts = _sc_spmm(h_n, P_att2, W_att2, nch_att2, nv_pad)
    rs2_parts = _sc_rowsum(P_att2, W_att2, nch_att2, nv_pad)

    h_v = _tc(_div_bn_tanh_kernel,
              jax.ShapeDtypeStruct((N_v, D), jnp.float32),
              pooled2_parts[:, :N_v], rs2_parts[:, :N_v],
              r1(p['bn1_g']), r1(p['bn1_b']))
    out = _gru(h, h_v,
               (p['gru_Wiz'], r1(p['gru_biz']), p['gru_Whz'], r1(p['gru_bhz']),
                p['gru_Wir'], r1(p['gru_bir']), p['gru_Whr'], r1(p['gru_bhr']),
                p['gru_Win'], r1(p['gru_bin']), p['gru_Whn'], r1(p['gru_bhn'])))
    return out
